# baseline (speedup 1.0000x reference)
import functools

import jax
import jax.numpy as jnp
from jax import lax
from jax.experimental import pallas as pl
from jax.experimental.pallas import tpu as pltpu
from jax.experimental.pallas import tpu_sc as plsc

_ROWS = 4096
_COLS = 4096
_BLOCK = 512

_NC = 2
_NS = 16
_NW = _NC * _NS
_R_SC = 512
_W = _R_SC // _NW
_CH = 8
_NCH = _W // _CH
_JV = _COLS // 16


def _tc_body(x_ref, out_ref):
    i = pl.program_id(0)
    m = (x_ref[...] != 0.0).astype(jnp.float32)
    sel = lax.broadcasted_iota(jnp.int32, (2, _BLOCK), 0) == 0
    lane = lax.broadcasted_iota(jnp.int32, (2, _BLOCK), 1).astype(jnp.float32)
    w = jnp.where(sel, lane, 1.0)
    r = lax.dot_general(w, m, (((1,), (0,)), ((), ())),
                        preferred_element_type=jnp.float32)
    ri = r.astype(jnp.int32)
    col_ids = lax.broadcasted_iota(jnp.int32, (1, _COLS), 1)
    s_local = jnp.sum(ri[0:1])
    nnz = jnp.sum(ri[1:2])
    p_row = s_local + (_R_SC + i * _BLOCK) * nnz
    p_col = jnp.sum(ri[1:2] * col_ids)

    @pl.when(i == 0)
    def _init():
        out_ref[0] = 0
        out_ref[1] = 0

    out_ref[0] += p_row
    out_ref[1] += p_col


def _tc_part(x):
    return pl.pallas_call(
        _tc_body,
        grid=((_ROWS - _R_SC) // _BLOCK,),
        in_specs=[pl.BlockSpec((_BLOCK, _COLS), lambda i: (i + _R_SC // _BLOCK, 0))],
        out_specs=pl.BlockSpec(memory_space=pltpu.SMEM),
        out_shape=jax.ShapeDtypeStruct((2,), jnp.int32),
    )(x)


def _sc_chunk(buf, base, rowpart, colpart):
    def jbody(j, carry):
        rowpart, colpart = carry
        col0 = j * 16
        colcnt = jnp.zeros((16,), jnp.int32)
        rweight = jnp.zeros((16,), jnp.int32)
        for r in range(_CH):
            xv = buf[r, pl.ds(col0, 16)]
            one = jnp.where(xv == 0.0, jnp.int32(0), jnp.int32(1))
            colcnt = colcnt + one
            if r:
                rweight = rweight + r * one
        rowpart = rowpart + base * colcnt + rweight
        colpart = colpart + colcnt * (lax.iota(jnp.int32, 16) + col0)
        return rowpart, colpart

    return lax.fori_loop(0, _JV, jbody, (rowpart, colpart))


def _sc_kernel(x_hbm, out_hbm, buf0, buf1, ovec, sem0, sem1):
    wid = lax.axis_index("s") * _NC + lax.axis_index("c")
    row0 = wid * _W
    bufs = (buf0, buf1)
    sems = (sem0, sem1)
    copies = [None] * _NCH
    copies[0] = pltpu.async_copy(x_hbm.at[pl.ds(row0, _CH)], buf0, sem0)
    rowpart = jnp.zeros((16,), jnp.int32)
    colpart = jnp.zeros((16,), jnp.int32)
    for c in range(_NCH):
        if c + 1 < _NCH:
            copies[c + 1] = pltpu.async_copy(
                x_hbm.at[pl.ds(row0 + (c + 1) * _CH, _CH)],
                bufs[(c + 1) % 2], sems[(c + 1) % 2])
        copies[c].wait()
        rowpart, colpart = _sc_chunk(bufs[c % 2], row0 + c * _CH,
                                     rowpart, colpart)
    ovec[0] = rowpart
    ovec[1] = colpart
    pltpu.sync_copy(ovec, out_hbm.at[wid])


def _sc_part(x):
    mesh = plsc.VectorSubcoreMesh(core_axis_name="c", subcore_axis_name="s")
    run = functools.partial(
        pl.kernel,
        mesh=mesh,
        out_type=jax.ShapeDtypeStruct((_NW, 2, 16), jnp.int32),
        scratch_types=[
            pltpu.VMEM((_CH, _COLS), jnp.float32),
            pltpu.VMEM((_CH, _COLS), jnp.float32),
            pltpu.VMEM((2, 16), jnp.int32),
            pltpu.SemaphoreType.DMA,
            pltpu.SemaphoreType.DMA,
        ],
    )(_sc_kernel)
    return run(x)


def kernel(inputs):
    sc = _sc_part(inputs)
    tc = _tc_part(inputs)
    return tc + jnp.sum(sc, axis=(0, 2))


# speedup vs baseline: 1.8253x; 1.8253x over previous
import jax
import jax.numpy as jnp
from jax import lax
from jax.experimental import pallas as pl
from jax.experimental.pallas import tpu as pltpu

_ROWS = 4096
_COLS = 4096
_BLOCK = 256
_STEPS = 8
_HALF = _STEPS * _BLOCK


def _partials(x_ref, row_off):
    m = (x_ref[...] != 0.0).astype(jnp.float32)
    sel = lax.broadcasted_iota(jnp.int32, (2, _BLOCK), 0) == 0
    lane = lax.broadcasted_iota(jnp.int32, (2, _BLOCK), 1).astype(jnp.float32)
    w = jnp.where(sel, lane, 1.0)
    r = lax.dot_general(w, m, (((1,), (0,)), ((), ())),
                        preferred_element_type=jnp.float32)
    ri = r.astype(jnp.int32)
    col_ids = lax.broadcasted_iota(jnp.int32, (1, _COLS), 1)
    nnz = jnp.sum(ri[1:2])
    p_row = jnp.sum(ri[0:1]) + row_off * nnz
    p_col = jnp.sum(ri[1:2] * col_ids)
    return p_row, p_col


def _body(a_ref, b_ref, out_ref):
    i = pl.program_id(0)
    pr_a, pc_a = _partials(a_ref, i * _BLOCK)
    pr_b, pc_b = _partials(b_ref, _HALF + i * _BLOCK)

    @pl.when(i == 0)
    def _init():
        out_ref[0] = 0
        out_ref[1] = 0

    out_ref[0] += pr_a + pr_b
    out_ref[1] += pc_a + pc_b


def kernel(inputs):
    return pl.pallas_call(
        _body,
        grid=(_STEPS,),
        in_specs=[
            pl.BlockSpec((_BLOCK, _COLS), lambda i: (i, 0)),
            pl.BlockSpec((_BLOCK, _COLS), lambda i: (i + _STEPS, 0)),
        ],
        out_specs=pl.BlockSpec(memory_space=pltpu.SMEM),
        out_shape=jax.ShapeDtypeStruct((2,), jnp.int32),
    )(inputs, inputs)
